# baseline (speedup 1.0000x reference)
import functools

import jax
import jax.numpy as jnp
from jax.experimental import pallas as pl
from jax.experimental.pallas import tpu as pltpu


def _stats_kernel(xa_ref, xb_ref, sum_ref, sq_ref):
    xa = xa_ref[...]
    xb = xb_ref[...]
    sum_ref[...] = (jnp.sum(xa, axis=0, keepdims=True)
                    + jnp.sum(xb, axis=0, keepdims=True))
    sq_ref[...] = (jnp.sum(xa * xa, axis=0, keepdims=True)
                   + jnp.sum(xb * xb, axis=0, keepdims=True))


def _fused_kernel(xa_ref, xb_ref, ra_ref, rb_ref, psum_ref, psq_ref,
                  gamma_ref, beta_ref, w_ref, o_ref, *, count, eps):
    total = jnp.sum(psum_ref[...], axis=0)
    total_sq = jnp.sum(psq_ref[...], axis=0)
    inv_count = 1.0 / count
    mean = total * inv_count
    var = total_sq * inv_count - mean * mean
    inv_std = jax.lax.rsqrt(var + eps)
    scale = gamma_ref[...] * inv_std
    shift = beta_ref[...] - mean * scale

    w = w_ref[...].astype(jnp.bfloat16)
    half = xa_ref.shape[0]
    ya = jnp.maximum(xa_ref[...] * scale + shift + ra_ref[...], 0.0)
    o_ref[:half, :] = jax.lax.dot_general(
        ya.astype(jnp.bfloat16), w, (((1,), (1,)), ((), ())),
        preferred_element_type=jnp.float32)
    yb = jnp.maximum(xb_ref[...] * scale + shift + rb_ref[...], 0.0)
    o_ref[half:, :] = jax.lax.dot_general(
        yb.astype(jnp.bfloat16), w, (((1,), (1,)), ((), ())),
        preferred_element_type=jnp.float32)


@functools.partial(jax.jit, static_argnames=("sblk",))
def _forward(x33, x26, gamma, beta, conv_w, *, sblk=3136):
    N, Cin, H, W = x33.shape
    Cout = conv_w.shape[0]
    S = H * W

    x = x33.transpose(0, 2, 3, 1).reshape(N, S, Cin)
    r = x26.transpose(0, 2, 3, 1).reshape(N, S, Cin)

    psum, psq = pl.pallas_call(
        _stats_kernel,
        out_shape=(
            jax.ShapeDtypeStruct((N // 2, 1, Cin), jnp.float32),
            jax.ShapeDtypeStruct((N // 2, 1, Cin), jnp.float32),
        ),
        grid=(N // 2,),
        in_specs=[
            pl.BlockSpec((None, S, Cin), lambda n: (2 * n, 0, 0)),
            pl.BlockSpec((None, S, Cin), lambda n: (2 * n + 1, 0, 0)),
        ],
        out_specs=(
            pl.BlockSpec((None, 1, Cin), lambda n: (n, 0, 0)),
            pl.BlockSpec((None, 1, Cin), lambda n: (n, 0, 0)),
        ),
        compiler_params=pltpu.CompilerParams(
            dimension_semantics=("parallel",)),
    )(x, x)

    w = conv_w.reshape(Cout, Cin)
    g2 = gamma.reshape(1, Cin)
    b2 = beta.reshape(1, Cin)

    ns = pl.cdiv(S, sblk)
    half = sblk // 2
    out = pl.pallas_call(
        functools.partial(_fused_kernel, count=N * S, eps=1e-5),
        out_shape=jax.ShapeDtypeStruct((N, S, Cout), jnp.float32),
        grid=(N, ns),
        in_specs=[
            pl.BlockSpec((None, half, Cin), lambda n, j: (n, 2 * j, 0)),
            pl.BlockSpec((None, half, Cin), lambda n, j: (n, 2 * j + 1, 0)),
            pl.BlockSpec((None, half, Cin), lambda n, j: (n, 2 * j, 0)),
            pl.BlockSpec((None, half, Cin), lambda n, j: (n, 2 * j + 1, 0)),
            pl.BlockSpec((N // 2, 1, Cin), lambda n, j: (0, 0, 0)),
            pl.BlockSpec((N // 2, 1, Cin), lambda n, j: (0, 0, 0)),
            pl.BlockSpec((1, Cin), lambda n, j: (0, 0)),
            pl.BlockSpec((1, Cin), lambda n, j: (0, 0)),
            pl.BlockSpec((Cout, Cin), lambda n, j: (0, 0)),
        ],
        out_specs=pl.BlockSpec((None, sblk, Cout), lambda n, j: (n, j, 0)),
        compiler_params=pltpu.CompilerParams(
            dimension_semantics=("parallel", "parallel")),
    )(x, x, r, r, psum, psq, g2, b2, w)
    return out.reshape(N, H, W, Cout).transpose(0, 3, 1, 2)


def kernel(x33, x26, gamma, beta, conv_w):
    return _forward(x33, x26, gamma, beta, conv_w)


# speedup vs baseline: 1.0174x; 1.0174x over previous
import functools

import jax
import jax.numpy as jnp
from jax.experimental import pallas as pl
from jax.experimental.pallas import tpu as pltpu


def _stats_kernel(xa_ref, xb_ref, sum_ref, sq_ref):
    xa = xa_ref[...]
    xb = xb_ref[...]
    sum_ref[...] = (jnp.sum(xa, axis=0, keepdims=True)
                    + jnp.sum(xb, axis=0, keepdims=True))
    sq_ref[...] = (jnp.sum(xa * xa, axis=0, keepdims=True)
                   + jnp.sum(xb * xb, axis=0, keepdims=True))


def _fused_kernel(x_ref, r_ref, psum_ref, psq_ref, gamma_ref, beta_ref,
                  w_ref, o_ref, *, count, eps):
    total = jnp.sum(psum_ref[...], axis=0)
    total_sq = jnp.sum(psq_ref[...], axis=0)
    inv_count = 1.0 / count
    mean = total * inv_count
    var = total_sq * inv_count - mean * mean
    inv_std = jax.lax.rsqrt(var + eps)
    scale = gamma_ref[...] * inv_std
    shift = beta_ref[...] - mean * scale

    y = jnp.maximum(x_ref[...] * scale + shift + r_ref[...], 0.0)
    o_ref[...] = jax.lax.dot_general(
        y.astype(jnp.bfloat16), w_ref[...].astype(jnp.bfloat16),
        (((1,), (1,)), ((), ())),
        preferred_element_type=jnp.float32)


@functools.partial(jax.jit, static_argnames=("sblk",))
def _forward(x33, x26, gamma, beta, conv_w, *, sblk=3136):
    N, Cin, H, W = x33.shape
    Cout = conv_w.shape[0]
    S = H * W

    x = x33.transpose(0, 2, 3, 1).reshape(N, S, Cin)
    r = x26.transpose(0, 2, 3, 1).reshape(N, S, Cin)

    psum, psq = pl.pallas_call(
        _stats_kernel,
        out_shape=(
            jax.ShapeDtypeStruct((N // 2, 1, Cin), jnp.float32),
            jax.ShapeDtypeStruct((N // 2, 1, Cin), jnp.float32),
        ),
        grid=(N // 2,),
        in_specs=[
            pl.BlockSpec((None, S, Cin), lambda n: (2 * n, 0, 0)),
            pl.BlockSpec((None, S, Cin), lambda n: (2 * n + 1, 0, 0)),
        ],
        out_specs=(
            pl.BlockSpec((None, 1, Cin), lambda n: (n, 0, 0)),
            pl.BlockSpec((None, 1, Cin), lambda n: (n, 0, 0)),
        ),
        compiler_params=pltpu.CompilerParams(
            dimension_semantics=("parallel",)),
    )(x, x)

    w = conv_w.reshape(Cout, Cin)
    g2 = gamma.reshape(1, Cin)
    b2 = beta.reshape(1, Cin)

    ns = pl.cdiv(S, sblk)
    out = pl.pallas_call(
        functools.partial(_fused_kernel, count=N * S, eps=1e-5),
        out_shape=jax.ShapeDtypeStruct((N, S, Cout), jnp.float32),
        grid=(N, ns),
        in_specs=[
            pl.BlockSpec((None, sblk, Cin), lambda n, j: (n, j, 0)),
            pl.BlockSpec((None, sblk, Cin), lambda n, j: (n, j, 0)),
            pl.BlockSpec((N // 2, 1, Cin), lambda n, j: (0, 0, 0)),
            pl.BlockSpec((N // 2, 1, Cin), lambda n, j: (0, 0, 0)),
            pl.BlockSpec((1, Cin), lambda n, j: (0, 0)),
            pl.BlockSpec((1, Cin), lambda n, j: (0, 0)),
            pl.BlockSpec((Cout, Cin), lambda n, j: (0, 0)),
        ],
        out_specs=pl.BlockSpec((None, sblk, Cout), lambda n, j: (n, j, 0)),
        compiler_params=pltpu.CompilerParams(
            dimension_semantics=("parallel", "parallel")),
    )(x, r, psum, psq, g2, b2, w)
    return out.reshape(N, H, W, Cout).transpose(0, 3, 1, 2)


def kernel(x33, x26, gamma, beta, conv_w):
    return _forward(x33, x26, gamma, beta, conv_w)
